# baseline (speedup 1.0000x reference)
import functools

import jax
import jax.numpy as jnp
from jax import lax
from jax.experimental import pallas as pl
from jax.experimental.pallas import tpu as pltpu
from jax.experimental.pallas import tpu_sc as plsc

N = 10000
E = 640000
EH = 128
RH = 32
NREL = 1000

NC = 2
NS = 16
NW = NC * NS
L = 16

K = 128
NJ = 10
CHUNK = NJ * K
NCHUNK = 16
NBUF = 4
E_PER = NCHUNK * CHUNK
E_PAD = NW * E_PER

NP = 10240
NRELP = 1024

_MESH = dict(core_axis_name="c", subcore_axis_name="s", num_cores=NC,
             num_subcores=NS)


def _sc_edge_pass(mode, d, n_out, np_out, n_u, n_v, has_feat, has_sacc,
                  w_out, n_in, uv_same=False, i1_is_gidx=False):
    out_type = []
    if has_feat:
        out_type.append(jax.ShapeDtypeStruct((NC * np_out, d), jnp.float32))
    if has_sacc:
        out_type.append(jax.ShapeDtypeStruct((NC * np_out, L), jnp.float32))
    if w_out:
        out_type.append(jax.ShapeDtypeStruct((E_PAD,), jnp.float32))

    scratch = {}
    if has_feat:
        scratch["facc"] = pltpu.VMEM_SHARED((np_out, d), jnp.float32)
        scratch["gidx_v"] = pltpu.VMEM((CHUNK,), jnp.int32)
        scratch["rows_v"] = pltpu.VMEM((NBUF, K, d), jnp.float32)
    if has_sacc:
        scratch["sacc"] = pltpu.VMEM_SHARED((np_out, L), jnp.float32)
        scratch["brows_v"] = pltpu.VMEM((K, L), jnp.float32)
    scratch["sidx_v"] = pltpu.VMEM((NJ, K), jnp.int32)
    if mode in ("gat", "gcn"):
        if not i1_is_gidx:
            scratch["i1_v"] = pltpu.VMEM((CHUNK,), jnp.int32)
        scratch["i2_v"] = pltpu.VMEM((CHUNK,), jnp.int32)
        scratch["u_v"] = pltpu.VMEM((n_u,), jnp.float32)
        if not uv_same:
            scratch["v_v"] = pltpu.VMEM((n_v,), jnp.float32)
    if mode != "ones":
        scratch["w_v"] = pltpu.VMEM((CHUNK,), jnp.float32)
    scratch["sem"] = pltpu.SemaphoreType.DMA
    if has_feat:
        for _b in range(1, NBUF):
            scratch[f"sem{_b}"] = pltpu.SemaphoreType.DMA
        for _b in range(NBUF):
            scratch[f"ssem{_b}"] = pltpu.SemaphoreType.DMA
    snames = list(scratch.keys())

    def body(*refs):
        nin = 0
        args = {}
        if has_feat:
            args["x_hbm"] = refs[nin]; nin += 1
            args["gidx_hbm"] = refs[nin]; nin += 1
        args["sidx_hbm"] = refs[nin]; nin += 1
        if mode in ("gat", "gcn"):
            if not i1_is_gidx:
                args["i1_hbm"] = refs[nin]; nin += 1
            args["i2_hbm"] = refs[nin]; nin += 1
            args["u_hbm"] = refs[nin]; nin += 1
            if not uv_same:
                args["v_hbm"] = refs[nin]; nin += 1
        if mode == "load":
            args["wsrc_hbm"] = refs[nin]; nin += 1
        if has_feat:
            args["facc_out"] = refs[nin]; nin += 1
        if has_sacc:
            args["sacc_out"] = refs[nin]; nin += 1
        if w_out:
            args["w_hbm"] = refs[nin]; nin += 1
        for nm, r in zip(snames, refs[nin:]):
            args[nm] = r
        if mode in ("gat", "gcn"):
            if i1_is_gidx:
                args["i1_v"] = args["gidx_v"]
            if uv_same:
                args["v_v"] = args["u_v"]

        c = lax.axis_index("c")
        s = lax.axis_index("s")
        wid = c * NS + s

        if mode in ("gat", "gcn"):
            pltpu.sync_copy(args["u_hbm"], args["u_v"])
            if not uv_same:
                pltpu.sync_copy(args["v_hbm"], args["v_v"])

        zero16 = jnp.zeros((L,), jnp.float32)
        rows_per = np_out // NS
        base_rows = s * rows_per

        def zero_buf(zref, width):
            def zrow(i, _):
                for dd in range(width // L):
                    zref[i, pl.ds(dd * L, L)] = zero16
                return 0
            lax.fori_loop(0, K, zrow, 0)

        def zero_acc(zref, acc):
            zr = min(rows_per, K)
            def zstep(r, _):
                pltpu.sync_copy(zref.at[pl.ds(0, zr)],
                                acc.at[pl.ds(base_rows + r * zr, zr)])
                return 0
            lax.fori_loop(0, rows_per // zr, zstep, 0)

        if has_feat:
            zero_buf(args["rows_v"].at[0], d)
            zero_acc(args["rows_v"].at[0], args["facc"])
        if has_sacc:
            zero_buf(args["brows_v"], L)
            zero_acc(args["brows_v"], args["sacc"])
        plsc.subcore_barrier()

        if mode == "ones":
            one16 = jnp.full((L,), 1.0, jnp.float32)

            def orow(i, _):
                args["brows_v"][i, :] = one16
                return 0
            lax.fori_loop(0, K, orow, 0)

        sems = [args.get("sem")] + [args.get(f"sem{b}")
                                    for b in range(1, NBUF)]
        ssems = [args.get(f"ssem{b}") for b in range(NBUF)]

        def chunk_body(ci, _):
            ebase = (wid * NCHUNK + ci) * CHUNK
            rbase = (wid * NCHUNK + ci) * NJ
            if has_feat:
                pltpu.sync_copy(args["gidx_hbm"].at[pl.ds(ebase, CHUNK)],
                                args["gidx_v"])

            def gather(j, b):
                return pltpu.async_copy(
                    args["x_hbm"].at[args["gidx_v"].at[pl.ds(j * K, K)]],
                    args["rows_v"].at[b], sems[b])

            cps = {}
            if has_feat:
                cps[0] = gather(0, 0)
            pltpu.sync_copy(args["sidx_hbm"].at[pl.ds(rbase, NJ)],
                            args["sidx_v"])
            if mode in ("gat", "gcn"):
                if not i1_is_gidx:
                    pltpu.sync_copy(args["i1_hbm"].at[pl.ds(ebase, CHUNK)],
                                    args["i1_v"])
                pltpu.sync_copy(args["i2_hbm"].at[pl.ds(ebase, CHUNK)],
                                args["i2_v"])

                def wstep(i, _):
                    off = i * L
                    idx1 = args["i1_v"][pl.ds(off, L)]
                    idx2 = args["i2_v"][pl.ds(off, L)]
                    a = plsc.load_gather(args["u_v"], [idx1])
                    b = plsc.load_gather(args["v_v"], [idx2])
                    if mode == "gat":
                        z = a + b
                        w16 = jnp.exp(jnp.maximum(z, 0.01 * z))
                    else:
                        w16 = a * b
                    args["w_v"][pl.ds(off, L)] = w16
                    return 0
                lax.fori_loop(0, CHUNK // L, wstep, 0)
            if mode == "load":
                pltpu.sync_copy(args["wsrc_hbm"].at[pl.ds(ebase, CHUNK)],
                                args["w_v"])
            if w_out:
                pltpu.sync_copy(args["w_v"],
                                args["w_hbm"].at[pl.ds(ebase, CHUNK)])

            scat = {}
            for j in range(NJ):
                b = j % NBUF
                if has_feat:
                    cps[b].wait()
                    if j + 1 < NJ:
                        b2 = (j + 1) % NBUF
                        if b2 in scat:
                            scat[b2].wait()
                        cps[b2] = gather(j + 1, b2)
                    rbuf = args["rows_v"].at[b]
                if mode != "ones":
                    def scale_grp(ii, _):
                        gbase = ii * L
                        w16 = args["w_v"][pl.ds(j * K + gbase, L)]
                        for l in range(L):
                            wb = lax.broadcast(w16[l], (L,))
                            row = gbase + l
                            if has_feat:
                                vals = [rbuf[row, pl.ds(dd * L, L)]
                                        for dd in range(d // L)]
                                for dd in range(d // L):
                                    rbuf[row, pl.ds(dd * L, L)] = (
                                        vals[dd] * wb)
                            if has_sacc:
                                args["brows_v"][row, :] = wb
                        return 0
                    lax.fori_loop(0, K // L, scale_grp, 0)
                row_idx = args["sidx_v"].at[j]
                if has_feat:
                    scat[b] = pltpu.async_copy(
                        rbuf, args["facc"].at[row_idx], ssems[b], add=True)
                if has_sacc:
                    pltpu.sync_copy(args["brows_v"],
                                    args["sacc"].at[row_idx], add=True)
            for b in list(scat):
                scat[b].wait()
            return 0
        lax.fori_loop(0, NCHUNK, chunk_body, 0)

        plsc.subcore_barrier()
        obase = c * np_out + base_rows
        if has_feat:
            pltpu.sync_copy(args["facc"].at[pl.ds(base_rows, rows_per)],
                            args["facc_out"].at[pl.ds(obase, rows_per)])
        if has_sacc:
            pltpu.sync_copy(args["sacc"].at[pl.ds(base_rows, rows_per)],
                            args["sacc_out"].at[pl.ds(obase, rows_per)])

    mesh = plsc.VectorSubcoreMesh(**_MESH)
    return pl.kernel(body, out_type=tuple(out_type), mesh=mesh,
                     scratch_types=list(scratch.values()),
                     compiler_params=pltpu.CompilerParams(
                         needs_layout_passes=False,
                         use_tc_tiling_on_sc=False))


R = 2048
GN = 5


def _tc_call(body, out_type):
    return pl.pallas_call(body, out_shape=out_type)


def _rows(w):
    return pl.BlockSpec((R, w), lambda i: (i, 0))


def _acc3(w):
    return pl.BlockSpec((NC, R, w), lambda i: (0, i, 0))


def _full(*shape):
    nd = len(shape)
    return pl.BlockSpec(shape, lambda i: (0,) * nd)


def _vec():
    return pl.BlockSpec((R,), lambda i: (i,))


def _inv0(s3):
    s0 = s3[0, :, 0] + s3[1, :, 0]
    return jnp.where(s0 > 0, 1.0 / s0, 0.0)[:, None]


def _dis_body(sacc_ref, out_ref):
    a = sacc_ref[...]
    deg = a[0, :, 0] + a[1, :, 0]
    out_ref[...] = jnp.where(deg > 0, lax.rsqrt(jnp.maximum(deg, 1e-30)), 0.0)


def _hw_body(xin_ref, gpa_ref, gpb_ref, w_ref, b_ref, out_ref):
    gpa = gpa_ref[...]
    gpb = gpb_ref[...]
    g = jax.nn.relu(jnp.concatenate([gpa[0] + gpa[1], gpb[0] + gpb[1]],
                                    axis=1))
    xin = xin_ref[...]
    gate = jax.nn.sigmoid(
        jnp.dot(xin, w_ref[...], preferred_element_type=jnp.float32)
        + b_ref[...])
    out_ref[...] = gate * g + (1.0 - gate) * xin


def _proj_body(x_ref, wh_ref, wt_ref, ah1_ref, ah2_ref, at1_ref, at2_ref,
               rah_ref, rat_ref,
               xrh_ref, xrt_ref, ph1_ref, ph2_ref, pt1_ref, pt2_ref,
               ehn_ref, etn_ref):
    x = x_ref[...]
    xrh = jnp.dot(x, wh_ref[...], preferred_element_type=jnp.float32)
    xrt = jnp.dot(x, wt_ref[...], preferred_element_type=jnp.float32)
    xrh_ref[...] = xrh
    xrt_ref[...] = xrt
    ph1_ref[...] = jnp.sum(xrh * ah1_ref[...], axis=1)
    ph2_ref[...] = jnp.sum(xrt * ah2_ref[...], axis=1)
    pt1_ref[...] = jnp.sum(xrh * at1_ref[...], axis=1)
    pt2_ref[...] = jnp.sum(xrt * at2_ref[...], axis=1)
    ehn_ref[...] = jnp.sum(x * rah_ref[...], axis=1)
    etn_ref[...] = jnp.sum(x * rat_ref[...], axis=1)


def _xr_body(fh_ref, sh_ref, ft_ref, st_ref, ar_ref, xr_ref, rp_ref):
    fh = fh_ref[...]
    ft = ft_ref[...]
    sh = sh_ref[...]
    st = st_ref[...]
    sh0 = sh[0, :NREL, 0] + sh[1, :NREL, 0]
    st0 = st[0, :NREL, 0] + st[1, :NREL, 0]
    inv_h = jnp.where(sh0 > 0, 1.0 / sh0, 0.0)[:, None]
    inv_t = jnp.where(st0 > 0, 1.0 / st0, 0.0)[:, None]
    xr = (fh[0, :NREL, :] + fh[1, :NREL, :]) * inv_h \
        + (ft[0, :NREL, :] + ft[1, :NREL, :]) * inv_t
    xr_ref[...] = xr
    rp_ref[...] = jnp.sum(xr * ar_ref[...], axis=1)


def _cat_body(x_ref, fh_ref, sh_ref, ft_ref, st_ref, ai_ref, aj_ref,
              xcat_ref, gi_ref, gj_ref):
    fh = fh_ref[...]
    ft = ft_ref[...]
    xeh = (fh[0] + fh[1]) * _inv0(sh_ref[...])
    xet = (ft[0] + ft[1]) * _inv0(st_ref[...])
    xcat = jnp.concatenate([x_ref[...], xeh, xet], axis=1)
    xcat_ref[...] = xcat
    gi_ref[...] = jnp.sum(xcat * ai_ref[...], axis=1)
    gj_ref[...] = jnp.sum(xcat * aj_ref[...], axis=1)


def _out_body(xcat_ref, fg_ref, sg_ref, out_ref):
    fg = fg_ref[...]
    xg = jax.nn.relu((fg[0] + fg[1]) * _inv0(sg_ref[...]))
    out_ref[...] = jnp.concatenate([xcat_ref[...], xg], axis=1)


def _padi(a, fill):
    return jnp.concatenate(
        [a, jnp.full((E_PAD - E,), fill, a.dtype)])


@jax.jit
def _run(x_e, edge_index, rel, edge_index_all,
         hw1_W, hw1_b, hw2_W, hw2_b,
         e2r_ah1, e2r_ah2, e2r_at1, e2r_at2, e2r_wh, e2r_wt,
         r2e_ah, r2e_at, r2e_ar, gat_ai, gat_aj):
    f32 = jnp.float32
    src_a = edge_index_all[0]
    dst_a = edge_index_all[1]
    h = edge_index[0]
    t = edge_index[1]

    src_a_g = _padi(src_a, 0)
    dst_a_g = _padi(dst_a, 0)
    dst_a_s = _padi(dst_a, N).reshape(E_PAD // K, K)
    h_g = _padi(h, 0)
    t_g = _padi(t, 0)
    rel_g = _padi(rel, 0)
    h_s = _padi(h, N).reshape(E_PAD // K, K)
    t_s = _padi(t, N).reshape(E_PAD // K, K)
    rel_s = _padi(rel, NREL).reshape(E_PAD // K, K)

    deg_pass = _sc_edge_pass("ones", 0, N, NP, 0, 0, False, True, False, 0)
    (sacc_deg,) = deg_pass(dst_a_s)
    dis = pl.pallas_call(
        _dis_body, grid=(GN,), in_specs=[_acc3(L)], out_specs=_vec(),
        out_shape=jax.ShapeDtypeStruct((N,), f32))(
        sacc_deg.reshape(NC, NP, L))

    eh2 = EH // 2
    gcn = _sc_edge_pass("gcn", eh2, N, NP, N, N, True, False, False, N,
                        uv_same=True, i1_is_gidx=True)
    (g1a,) = gcn(x_e[:, :eh2], src_a_g, dst_a_s, dst_a_g, dis)
    (g1b,) = gcn(x_e[:, eh2:], src_a_g, dst_a_s, dst_a_g, dis)
    hw_call = pl.pallas_call(
        _hw_body, grid=(GN,),
        in_specs=[_rows(EH), _acc3(eh2), _acc3(eh2), _full(EH, EH),
                  _full(1, EH)],
        out_specs=_rows(EH),
        out_shape=jax.ShapeDtypeStruct((N, EH), f32))
    x1 = hw_call(x_e, g1a.reshape(NC, NP, eh2), g1b.reshape(NC, NP, eh2),
                 hw1_W, hw1_b.reshape(1, EH))

    (g2a,) = gcn(x1[:, :eh2], src_a_g, dst_a_s, dst_a_g, dis)
    (g2b,) = gcn(x1[:, eh2:], src_a_g, dst_a_s, dst_a_g, dis)
    x = hw_call(x1, g2a.reshape(NC, NP, eh2), g2b.reshape(NC, NP, eh2),
                hw2_W, hw2_b.reshape(1, EH))

    outs = pl.pallas_call(
        _proj_body, grid=(GN,),
        in_specs=[_rows(EH), _full(EH, RH), _full(EH, RH)]
        + [_full(1, RH)] * 4 + [_full(1, EH)] * 2,
        out_specs=(_rows(RH), _rows(RH)) + (_vec(),) * 6,
        out_shape=(
            jax.ShapeDtypeStruct((N, RH), f32),
            jax.ShapeDtypeStruct((N, RH), f32),
        ) + (jax.ShapeDtypeStruct((N,), f32),) * 6,
    )(x, e2r_wh, e2r_wt,
      e2r_ah1.reshape(1, RH), e2r_ah2.reshape(1, RH),
      e2r_at1.reshape(1, RH), e2r_at2.reshape(1, RH),
      r2e_ah.reshape(1, EH), r2e_at.reshape(1, EH))
    xrh, xrt, ph1, ph2, pt1, pt2, ehn, etn = outs

    e2r = _sc_edge_pass("gat", RH, NREL, NRELP, N, N, True, True, False, N)
    fh, sh = e2r(xrh, h_g, rel_s, h_g, t_g, ph1, ph2)
    ft, st = e2r(xrt, t_g, rel_s, h_g, t_g, pt1, pt2)
    x_r, r_proj = _tc_call(_xr_body, (
        jax.ShapeDtypeStruct((NREL, RH), f32),
        jax.ShapeDtypeStruct((NREL,), f32),
    ))(fh.reshape(NC, NRELP, RH), sh.reshape(NC, NRELP, L),
       ft.reshape(NC, NRELP, RH), st.reshape(NC, NRELP, L),
       r2e_ar.reshape(1, RH))

    r2e = _sc_edge_pass("gat", RH, N, NP, N, NREL, True, True, False, NREL)
    fxh, sxh = r2e(x_r, rel_g, h_s, h_g, rel_g, ehn, r_proj)
    fxt, sxt = r2e(x_r, rel_g, t_s, t_g, rel_g, etn, r_proj)
    dcat = EH + 2 * RH
    xcat, gi, gj = pl.pallas_call(
        _cat_body, grid=(GN,),
        in_specs=[_rows(EH), _acc3(RH), _acc3(L), _acc3(RH), _acc3(L),
                  _full(1, dcat), _full(1, dcat)],
        out_specs=(_rows(dcat), _vec(), _vec()),
        out_shape=(
            jax.ShapeDtypeStruct((N, dcat), f32),
            jax.ShapeDtypeStruct((N,), f32),
            jax.ShapeDtypeStruct((N,), f32),
        ),
    )(x, fxh.reshape(NC, NP, RH), sxh.reshape(NC, NP, L),
      fxt.reshape(NC, NP, RH), sxt.reshape(NC, NP, L),
      gat_ai.reshape(1, dcat), gat_aj.reshape(1, dcat))

    fin_a = _sc_edge_pass("gat", 0, N, NP, N, N, False, True, True, 0)
    sg, w_all = fin_a(dst_a_s, dst_a_g, src_a_g, gi, gj)
    dh = dcat // 2
    fin_b = _sc_edge_pass("load", dh, N, NP, 0, 0, True, False, False, N)
    (fg0,) = fin_b(xcat[:, :dh], src_a_g, dst_a_s, w_all)
    (fg1,) = fin_b(xcat[:, dh:], src_a_g, dst_a_s, w_all)
    fg = jnp.concatenate([fg0.reshape(NC, NP, dh), fg1.reshape(NC, NP, dh)],
                         axis=2)

    return pl.pallas_call(
        _out_body, grid=(GN,),
        in_specs=[_rows(dcat), _acc3(dcat), _acc3(L)],
        out_specs=_rows(2 * dcat),
        out_shape=jax.ShapeDtypeStruct((N, 2 * dcat), f32))(
        xcat, fg, sg.reshape(NC, NP, L))


def kernel(x_e, edge_index, rel, edge_index_all, rel_all, hw1_W, hw1_b,
           hw2_W, hw2_b, e2r_ah1, e2r_ah2, e2r_at1, e2r_at2, e2r_wh,
           e2r_wt, r2e_ah, r2e_at, r2e_ar, gat_ai, gat_aj):
    return _run(x_e, edge_index, rel, edge_index_all,
                hw1_W, hw1_b, hw2_W, hw2_b,
                e2r_ah1, e2r_ah2, e2r_at1, e2r_at2, e2r_wh, e2r_wt,
                r2e_ah, r2e_at, r2e_ar, gat_ai, gat_aj)


# speedup vs baseline: 1.1183x; 1.1183x over previous
import functools

import jax
import jax.numpy as jnp
from jax import lax
from jax.experimental import pallas as pl
from jax.experimental.pallas import tpu as pltpu
from jax.experimental.pallas import tpu_sc as plsc

N = 10000
E = 640000
EH = 128
RH = 32
NREL = 1000

NC = 2
NS = 16
NW = NC * NS
L = 16

K = 128
NJ = 10
CHUNK = NJ * K
NCHUNK = 16
NBUF = 2
E_PER = NCHUNK * CHUNK
E_PAD = NW * E_PER

NP = 10240
NRELP = 1024

_MESH = dict(core_axis_name="c", subcore_axis_name="s", num_cores=NC,
             num_subcores=NS)


def _sc_edge_pass(mode, d, n_out, np_out, n_u, n_v, has_feat, has_sacc,
                  w_out, n_in, uv_same=False, i1_is_gidx=False):
    out_type = []
    if has_feat:
        out_type.append(jax.ShapeDtypeStruct((NC * np_out, d), jnp.float32))
    if has_sacc:
        out_type.append(jax.ShapeDtypeStruct((NC * np_out, L), jnp.float32))
    if w_out:
        out_type.append(jax.ShapeDtypeStruct((E_PAD,), jnp.float32))

    scratch = {}
    if has_feat:
        scratch["facc"] = pltpu.VMEM_SHARED((np_out, d), jnp.float32)
        scratch["gidx_v"] = pltpu.VMEM((CHUNK,), jnp.int32)
        scratch["rows_v"] = pltpu.VMEM((NBUF, K, d), jnp.float32)
    if has_sacc:
        scratch["sacc"] = pltpu.VMEM_SHARED((np_out, L), jnp.float32)
        scratch["brows_v"] = pltpu.VMEM((K, L), jnp.float32)
    scratch["sidx_v"] = pltpu.VMEM((NJ, K), jnp.int32)
    if mode in ("gat", "gcn"):
        if not i1_is_gidx:
            scratch["i1_v"] = pltpu.VMEM((CHUNK,), jnp.int32)
        scratch["i2_v"] = pltpu.VMEM((CHUNK,), jnp.int32)
        scratch["u_v"] = pltpu.VMEM((n_u,), jnp.float32)
        if not uv_same:
            scratch["v_v"] = pltpu.VMEM((n_v,), jnp.float32)
    if mode != "ones":
        scratch["w_v"] = pltpu.VMEM((CHUNK,), jnp.float32)
    scratch["sem"] = pltpu.SemaphoreType.DMA
    if has_feat:
        for _b in range(1, NBUF):
            scratch[f"sem{_b}"] = pltpu.SemaphoreType.DMA
        for _b in range(NBUF):
            scratch[f"ssem{_b}"] = pltpu.SemaphoreType.DMA
    snames = list(scratch.keys())

    def body(*refs):
        nin = 0
        args = {}
        if has_feat:
            args["x_hbm"] = refs[nin]; nin += 1
            args["gidx_hbm"] = refs[nin]; nin += 1
        args["sidx_hbm"] = refs[nin]; nin += 1
        if mode in ("gat", "gcn"):
            if not i1_is_gidx:
                args["i1_hbm"] = refs[nin]; nin += 1
            args["i2_hbm"] = refs[nin]; nin += 1
            args["u_hbm"] = refs[nin]; nin += 1
            if not uv_same:
                args["v_hbm"] = refs[nin]; nin += 1
        if mode == "load":
            args["wsrc_hbm"] = refs[nin]; nin += 1
        if has_feat:
            args["facc_out"] = refs[nin]; nin += 1
        if has_sacc:
            args["sacc_out"] = refs[nin]; nin += 1
        if w_out:
            args["w_hbm"] = refs[nin]; nin += 1
        for nm, r in zip(snames, refs[nin:]):
            args[nm] = r
        if mode in ("gat", "gcn"):
            if i1_is_gidx:
                args["i1_v"] = args["gidx_v"]
            if uv_same:
                args["v_v"] = args["u_v"]

        c = lax.axis_index("c")
        s = lax.axis_index("s")
        wid = c * NS + s

        if mode in ("gat", "gcn"):
            pltpu.sync_copy(args["u_hbm"], args["u_v"])
            if not uv_same:
                pltpu.sync_copy(args["v_hbm"], args["v_v"])

        zero16 = jnp.zeros((L,), jnp.float32)
        rows_per = np_out // NS
        base_rows = s * rows_per

        def zero_buf(zref, width):
            def zrow(i, _):
                for dd in range(width // L):
                    zref[i, pl.ds(dd * L, L)] = zero16
                return 0
            lax.fori_loop(0, K, zrow, 0)

        def zero_acc(zref, acc):
            zr = min(rows_per, K)
            def zstep(r, _):
                pltpu.sync_copy(zref.at[pl.ds(0, zr)],
                                acc.at[pl.ds(base_rows + r * zr, zr)])
                return 0
            lax.fori_loop(0, rows_per // zr, zstep, 0)

        if has_feat:
            zero_buf(args["rows_v"].at[0], d)
            zero_acc(args["rows_v"].at[0], args["facc"])
        if has_sacc:
            zero_buf(args["brows_v"], L)
            zero_acc(args["brows_v"], args["sacc"])
        plsc.subcore_barrier()

        if mode == "ones":
            one16 = jnp.full((L,), 1.0, jnp.float32)

            def orow(i, _):
                args["brows_v"][i, :] = one16
                return 0
            lax.fori_loop(0, K, orow, 0)

        sems = [args.get("sem")] + [args.get(f"sem{b}")
                                    for b in range(1, NBUF)]
        ssems = [args.get(f"ssem{b}") for b in range(NBUF)]

        def chunk_body(ci, _):
            ebase = (wid * NCHUNK + ci) * CHUNK
            rbase = (wid * NCHUNK + ci) * NJ
            if has_feat:
                pltpu.sync_copy(args["gidx_hbm"].at[pl.ds(ebase, CHUNK)],
                                args["gidx_v"])

            def gather(j, b):
                return pltpu.async_copy(
                    args["x_hbm"].at[args["gidx_v"].at[pl.ds(j * K, K)]],
                    args["rows_v"].at[b], sems[b])

            cps = {}
            if has_feat:
                cps[0] = gather(0, 0)
            pltpu.sync_copy(args["sidx_hbm"].at[pl.ds(rbase, NJ)],
                            args["sidx_v"])
            if mode in ("gat", "gcn"):
                if not i1_is_gidx:
                    pltpu.sync_copy(args["i1_hbm"].at[pl.ds(ebase, CHUNK)],
                                    args["i1_v"])
                pltpu.sync_copy(args["i2_hbm"].at[pl.ds(ebase, CHUNK)],
                                args["i2_v"])

                def wstep(i, _):
                    off = i * L
                    idx1 = args["i1_v"][pl.ds(off, L)]
                    idx2 = args["i2_v"][pl.ds(off, L)]
                    a = plsc.load_gather(args["u_v"], [idx1])
                    b = plsc.load_gather(args["v_v"], [idx2])
                    if mode == "gat":
                        z = a + b
                        w16 = jnp.exp(jnp.maximum(z, 0.01 * z))
                    else:
                        w16 = a * b
                    args["w_v"][pl.ds(off, L)] = w16
                    return 0
                lax.fori_loop(0, CHUNK // L, wstep, 0)
            if mode == "load":
                pltpu.sync_copy(args["wsrc_hbm"].at[pl.ds(ebase, CHUNK)],
                                args["w_v"])
            if w_out:
                pltpu.sync_copy(args["w_v"],
                                args["w_hbm"].at[pl.ds(ebase, CHUNK)])

            scat = {}
            for j in range(NJ):
                b = j % NBUF
                if has_feat:
                    cps[b].wait()
                    if j + 1 < NJ:
                        b2 = (j + 1) % NBUF
                        if b2 in scat:
                            scat[b2].wait()
                        cps[b2] = gather(j + 1, b2)
                    rbuf = args["rows_v"].at[b]
                if mode != "ones":
                    def scale_grp(ii, _):
                        gbase = ii * L
                        w16 = args["w_v"][pl.ds(j * K + gbase, L)]
                        for l in range(L):
                            wb = lax.broadcast(w16[l], (L,))
                            row = gbase + l
                            if has_feat:
                                vals = [rbuf[row, pl.ds(dd * L, L)]
                                        for dd in range(d // L)]
                                for dd in range(d // L):
                                    rbuf[row, pl.ds(dd * L, L)] = (
                                        vals[dd] * wb)
                            if has_sacc:
                                args["brows_v"][row, :] = wb
                        return 0
                    lax.fori_loop(0, K // L, scale_grp, 0)
                row_idx = args["sidx_v"].at[j]
                if has_feat:
                    scat[b] = pltpu.async_copy(
                        rbuf, args["facc"].at[row_idx], ssems[b], add=True)
                if has_sacc:
                    pltpu.sync_copy(args["brows_v"],
                                    args["sacc"].at[row_idx], add=True)
            for b in list(scat):
                scat[b].wait()
            return 0
        lax.fori_loop(0, NCHUNK, chunk_body, 0)

        plsc.subcore_barrier()
        obase = c * np_out + base_rows
        if has_feat:
            pltpu.sync_copy(args["facc"].at[pl.ds(base_rows, rows_per)],
                            args["facc_out"].at[pl.ds(obase, rows_per)])
        if has_sacc:
            pltpu.sync_copy(args["sacc"].at[pl.ds(base_rows, rows_per)],
                            args["sacc_out"].at[pl.ds(obase, rows_per)])

    mesh = plsc.VectorSubcoreMesh(**_MESH)
    return pl.kernel(body, out_type=tuple(out_type), mesh=mesh,
                     scratch_types=list(scratch.values()),
                     compiler_params=pltpu.CompilerParams(
                         needs_layout_passes=False,
                         use_tc_tiling_on_sc=False))


def _sc_r2e_dual():
    d = RH
    out_type = (
        jax.ShapeDtypeStruct((NC * NP, d), jnp.float32),
        jax.ShapeDtypeStruct((NC * NP, L), jnp.float32),
        jax.ShapeDtypeStruct((NC * NP, d), jnp.float32),
        jax.ShapeDtypeStruct((NC * NP, L), jnp.float32),
    )
    scratch = {
        "facc1": pltpu.VMEM_SHARED((NP, d), jnp.float32),
        "sacc1": pltpu.VMEM_SHARED((NP, L), jnp.float32),
        "facc2": pltpu.VMEM_SHARED((NP, d), jnp.float32),
        "sacc2": pltpu.VMEM_SHARED((NP, L), jnp.float32),
        "gidx_v": pltpu.VMEM((CHUNK,), jnp.int32),
        "h_v": pltpu.VMEM((CHUNK,), jnp.int32),
        "t_v": pltpu.VMEM((CHUNK,), jnp.int32),
        "sidx1_v": pltpu.VMEM((NJ, K), jnp.int32),
        "sidx2_v": pltpu.VMEM((NJ, K), jnp.int32),
        "u1_v": pltpu.VMEM((N,), jnp.float32),
        "u2_v": pltpu.VMEM((N,), jnp.float32),
        "v_v": pltpu.VMEM((NREL,), jnp.float32),
        "w1_v": pltpu.VMEM((CHUNK,), jnp.float32),
        "w2_v": pltpu.VMEM((CHUNK,), jnp.float32),
        "raw_v": pltpu.VMEM((NBUF, K, d), jnp.float32),
        "s1_v": pltpu.VMEM((NBUF, K, d), jnp.float32),
        "s2_v": pltpu.VMEM((NBUF, K, d), jnp.float32),
        "brows1_v": pltpu.VMEM((K, L), jnp.float32),
        "brows2_v": pltpu.VMEM((K, L), jnp.float32),
    }
    for _b in range(NBUF):
        scratch[f"gsem{_b}"] = pltpu.SemaphoreType.DMA
        scratch[f"s1sem{_b}"] = pltpu.SemaphoreType.DMA
        scratch[f"s2sem{_b}"] = pltpu.SemaphoreType.DMA
    snames = list(scratch.keys())

    def body(xr_hbm, rel_hbm, h2d_hbm, t2d_hbm, h_hbm, t_hbm,
             u1_hbm, u2_hbm, v_hbm, facc1_out, sacc1_out, facc2_out,
             sacc2_out, *srefs):
        a = dict(zip(snames, srefs))
        c = lax.axis_index("c")
        s = lax.axis_index("s")
        wid = c * NS + s
        pltpu.sync_copy(u1_hbm, a["u1_v"])
        pltpu.sync_copy(u2_hbm, a["u2_v"])
        pltpu.sync_copy(v_hbm, a["v_v"])

        zero16 = jnp.zeros((L,), jnp.float32)
        rows_per = NP // NS
        base_rows = s * rows_per

        def zero_buf(zref, width):
            def zrow(i, _):
                for dd in range(width // L):
                    zref[i, pl.ds(dd * L, L)] = zero16
                return 0
            lax.fori_loop(0, K, zrow, 0)

        def zero_acc(zref, acc):
            def zstep(r, _):
                pltpu.sync_copy(zref.at[pl.ds(0, K)],
                                acc.at[pl.ds(base_rows + r * K, K)])
                return 0
            lax.fori_loop(0, rows_per // K, zstep, 0)

        zero_buf(a["s1_v"].at[0], d)
        zero_buf(a["brows1_v"], L)
        zero_acc(a["s1_v"].at[0], a["facc1"])
        zero_acc(a["s1_v"].at[0], a["facc2"])
        zero_acc(a["brows1_v"], a["sacc1"])
        zero_acc(a["brows1_v"], a["sacc2"])
        plsc.subcore_barrier()

        gsems = [a[f"gsem{b}"] for b in range(NBUF)]
        s1sems = [a[f"s1sem{b}"] for b in range(NBUF)]
        s2sems = [a[f"s2sem{b}"] for b in range(NBUF)]

        def chunk_body(ci, _):
            ebase = (wid * NCHUNK + ci) * CHUNK
            rbase = (wid * NCHUNK + ci) * NJ
            pltpu.sync_copy(rel_hbm.at[pl.ds(ebase, CHUNK)], a["gidx_v"])

            def gather(j, b):
                return pltpu.async_copy(
                    xr_hbm.at[a["gidx_v"].at[pl.ds(j * K, K)]],
                    a["raw_v"].at[b], gsems[b])

            cps = {0: gather(0, 0)}
            pltpu.sync_copy(h2d_hbm.at[pl.ds(rbase, NJ)], a["sidx1_v"])
            pltpu.sync_copy(t2d_hbm.at[pl.ds(rbase, NJ)], a["sidx2_v"])
            pltpu.sync_copy(h_hbm.at[pl.ds(ebase, CHUNK)], a["h_v"])
            pltpu.sync_copy(t_hbm.at[pl.ds(ebase, CHUNK)], a["t_v"])

            def wstep(i, _):
                off = i * L
                ih = a["h_v"][pl.ds(off, L)]
                it = a["t_v"][pl.ds(off, L)]
                ir = a["gidx_v"][pl.ds(off, L)]
                a1 = plsc.load_gather(a["u1_v"], [ih])
                a2 = plsc.load_gather(a["u2_v"], [it])
                br = plsc.load_gather(a["v_v"], [ir])
                z1 = a1 + br
                z2 = a2 + br
                a["w1_v"][pl.ds(off, L)] = jnp.exp(
                    jnp.maximum(z1, 0.01 * z1))
                a["w2_v"][pl.ds(off, L)] = jnp.exp(
                    jnp.maximum(z2, 0.01 * z2))
                return 0
            lax.fori_loop(0, CHUNK // L, wstep, 0)

            sc1, sc2 = {}, {}
            for j in range(NJ):
                b = j % NBUF
                cps[b].wait()
                if j + 1 < NJ:
                    b2 = (j + 1) % NBUF
                    cps[b2] = gather(j + 1, b2)
                if b in sc1:
                    sc1[b].wait()
                    sc2[b].wait()
                rraw = a["raw_v"].at[b]
                rs1 = a["s1_v"].at[b]
                rs2 = a["s2_v"].at[b]

                def scale_grp(ii, _):
                    gbase = ii * L
                    w116 = a["w1_v"][pl.ds(j * K + gbase, L)]
                    w216 = a["w2_v"][pl.ds(j * K + gbase, L)]
                    for l in range(L):
                        wb1 = lax.broadcast(w116[l], (L,))
                        wb2 = lax.broadcast(w216[l], (L,))
                        row = gbase + l
                        vals = [rraw[row, pl.ds(dd * L, L)]
                                for dd in range(d // L)]
                        for dd in range(d // L):
                            rs1[row, pl.ds(dd * L, L)] = vals[dd] * wb1
                        for dd in range(d // L):
                            rs2[row, pl.ds(dd * L, L)] = vals[dd] * wb2
                        a["brows1_v"][row, :] = wb1
                        a["brows2_v"][row, :] = wb2
                    return 0
                lax.fori_loop(0, K // L, scale_grp, 0)

                i1 = a["sidx1_v"].at[j]
                i2 = a["sidx2_v"].at[j]
                sc1[b] = pltpu.async_copy(rs1, a["facc1"].at[i1],
                                          s1sems[b], add=True)
                sc2[b] = pltpu.async_copy(rs2, a["facc2"].at[i2],
                                          s2sems[b], add=True)
                pltpu.sync_copy(a["brows1_v"], a["sacc1"].at[i1], add=True)
                pltpu.sync_copy(a["brows2_v"], a["sacc2"].at[i2], add=True)
            for b in list(sc1):
                sc1[b].wait()
                sc2[b].wait()
            return 0
        lax.fori_loop(0, NCHUNK, chunk_body, 0)

        plsc.subcore_barrier()
        obase = c * NP + base_rows
        sl_s = pl.ds(base_rows, rows_per)
        sl_o = pl.ds(obase, rows_per)
        pltpu.sync_copy(a["facc1"].at[sl_s], facc1_out.at[sl_o])
        pltpu.sync_copy(a["sacc1"].at[sl_s], sacc1_out.at[sl_o])
        pltpu.sync_copy(a["facc2"].at[sl_s], facc2_out.at[sl_o])
        pltpu.sync_copy(a["sacc2"].at[sl_s], sacc2_out.at[sl_o])

    mesh = plsc.VectorSubcoreMesh(**_MESH)
    return pl.kernel(body, out_type=out_type, mesh=mesh,
                     scratch_types=list(scratch.values()),
                     compiler_params=pltpu.CompilerParams(
                         needs_layout_passes=False,
                         use_tc_tiling_on_sc=False))


R = 2048
GN = 5


def _tc_call(body, out_type):
    return pl.pallas_call(body, out_shape=out_type)


def _rows(w):
    return pl.BlockSpec((R, w), lambda i: (i, 0))


def _acc3(w):
    return pl.BlockSpec((NC, R, w), lambda i: (0, i, 0))


def _full(*shape):
    nd = len(shape)
    return pl.BlockSpec(shape, lambda i: (0,) * nd)


def _vec():
    return pl.BlockSpec((R,), lambda i: (i,))


def _inv0(s3):
    s0 = s3[0, :, 0] + s3[1, :, 0]
    return jnp.where(s0 > 0, 1.0 / s0, 0.0)[:, None]


def _dis_body(sacc_ref, out_ref):
    a = sacc_ref[...]
    deg = a[0, :, 0] + a[1, :, 0]
    out_ref[...] = jnp.where(deg > 0, lax.rsqrt(jnp.maximum(deg, 1e-30)), 0.0)


def _hw_body(xin_ref, gp_ref, w_ref, b_ref, out_ref):
    gp = gp_ref[...]
    g = jax.nn.relu(gp[0] + gp[1])
    xin = xin_ref[...]
    gate = jax.nn.sigmoid(
        jnp.dot(xin, w_ref[...], preferred_element_type=jnp.float32)
        + b_ref[...])
    out_ref[...] = gate * g + (1.0 - gate) * xin


def _proj_body(x_ref, wh_ref, wt_ref, ah1_ref, ah2_ref, at1_ref, at2_ref,
               rah_ref, rat_ref,
               xrh_ref, xrt_ref, ph1_ref, ph2_ref, pt1_ref, pt2_ref,
               ehn_ref, etn_ref):
    x = x_ref[...]
    xrh = jnp.dot(x, wh_ref[...], preferred_element_type=jnp.float32)
    xrt = jnp.dot(x, wt_ref[...], preferred_element_type=jnp.float32)
    xrh_ref[...] = xrh
    xrt_ref[...] = xrt
    ph1_ref[...] = jnp.sum(xrh * ah1_ref[...], axis=1)
    ph2_ref[...] = jnp.sum(xrt * ah2_ref[...], axis=1)
    pt1_ref[...] = jnp.sum(xrh * at1_ref[...], axis=1)
    pt2_ref[...] = jnp.sum(xrt * at2_ref[...], axis=1)
    ehn_ref[...] = jnp.sum(x * rah_ref[...], axis=1)
    etn_ref[...] = jnp.sum(x * rat_ref[...], axis=1)


def _xr_body(fh_ref, sh_ref, ft_ref, st_ref, ar_ref, xr_ref, rp_ref):
    fh = fh_ref[...]
    ft = ft_ref[...]
    sh = sh_ref[...]
    st = st_ref[...]
    sh0 = sh[0, :NREL, 0] + sh[1, :NREL, 0]
    st0 = st[0, :NREL, 0] + st[1, :NREL, 0]
    inv_h = jnp.where(sh0 > 0, 1.0 / sh0, 0.0)[:, None]
    inv_t = jnp.where(st0 > 0, 1.0 / st0, 0.0)[:, None]
    xr = (fh[0, :NREL, :] + fh[1, :NREL, :]) * inv_h \
        + (ft[0, :NREL, :] + ft[1, :NREL, :]) * inv_t
    xr_ref[...] = xr
    rp_ref[...] = jnp.sum(xr * ar_ref[...], axis=1)


def _cat_body(x_ref, fh_ref, sh_ref, ft_ref, st_ref, ai_ref, aj_ref,
              xcat_ref, gi_ref, gj_ref):
    fh = fh_ref[...]
    ft = ft_ref[...]
    xeh = (fh[0] + fh[1]) * _inv0(sh_ref[...])
    xet = (ft[0] + ft[1]) * _inv0(st_ref[...])
    xcat = jnp.concatenate([x_ref[...], xeh, xet], axis=1)
    xcat_ref[...] = xcat
    gi_ref[...] = jnp.sum(xcat * ai_ref[...], axis=1)
    gj_ref[...] = jnp.sum(xcat * aj_ref[...], axis=1)


def _out_body(xcat_ref, fg_ref, sg_ref, out_ref):
    fg = fg_ref[...]
    xg = jax.nn.relu((fg[0] + fg[1]) * _inv0(sg_ref[...]))
    out_ref[...] = jnp.concatenate([xcat_ref[...], xg], axis=1)


def _padi(a, fill):
    return jnp.concatenate(
        [a, jnp.full((E_PAD - E,), fill, a.dtype)])


@jax.jit
def _run(x_e, edge_index, rel, edge_index_all,
         hw1_W, hw1_b, hw2_W, hw2_b,
         e2r_ah1, e2r_ah2, e2r_at1, e2r_at2, e2r_wh, e2r_wt,
         r2e_ah, r2e_at, r2e_ar, gat_ai, gat_aj):
    f32 = jnp.float32
    src_a = edge_index_all[0]
    dst_a = edge_index_all[1]
    h = edge_index[0]
    t = edge_index[1]

    src_a_g = _padi(src_a, 0)
    dst_a_g = _padi(dst_a, 0)
    dst_a_s = _padi(dst_a, N).reshape(E_PAD // K, K)
    h_g = _padi(h, 0)
    t_g = _padi(t, 0)
    rel_g = _padi(rel, 0)
    h_s = _padi(h, N).reshape(E_PAD // K, K)
    t_s = _padi(t, N).reshape(E_PAD // K, K)
    rel_s = _padi(rel, NREL).reshape(E_PAD // K, K)

    deg_pass = _sc_edge_pass("ones", 0, N, NP, 0, 0, False, True, False, 0)
    (sacc_deg,) = deg_pass(dst_a_s)
    dis = pl.pallas_call(
        _dis_body, grid=(GN,), in_specs=[_acc3(L)], out_specs=_vec(),
        out_shape=jax.ShapeDtypeStruct((N,), f32))(
        sacc_deg.reshape(NC, NP, L))

    gcn = _sc_edge_pass("gcn", EH, N, NP, N, N, True, False, False, N,
                        uv_same=True, i1_is_gidx=True)
    (g1,) = gcn(x_e, src_a_g, dst_a_s, dst_a_g, dis)
    hw_call = pl.pallas_call(
        _hw_body, grid=(GN,),
        in_specs=[_rows(EH), _acc3(EH), _full(EH, EH), _full(1, EH)],
        out_specs=_rows(EH),
        out_shape=jax.ShapeDtypeStruct((N, EH), f32))
    x1 = hw_call(x_e, g1.reshape(NC, NP, EH), hw1_W, hw1_b.reshape(1, EH))

    (g2,) = gcn(x1, src_a_g, dst_a_s, dst_a_g, dis)
    x = hw_call(x1, g2.reshape(NC, NP, EH), hw2_W, hw2_b.reshape(1, EH))

    outs = pl.pallas_call(
        _proj_body, grid=(GN,),
        in_specs=[_rows(EH), _full(EH, RH), _full(EH, RH)]
        + [_full(1, RH)] * 4 + [_full(1, EH)] * 2,
        out_specs=(_rows(RH), _rows(RH)) + (_vec(),) * 6,
        out_shape=(
            jax.ShapeDtypeStruct((N, RH), f32),
            jax.ShapeDtypeStruct((N, RH), f32),
        ) + (jax.ShapeDtypeStruct((N,), f32),) * 6,
    )(x, e2r_wh, e2r_wt,
      e2r_ah1.reshape(1, RH), e2r_ah2.reshape(1, RH),
      e2r_at1.reshape(1, RH), e2r_at2.reshape(1, RH),
      r2e_ah.reshape(1, EH), r2e_at.reshape(1, EH))
    xrh, xrt, ph1, ph2, pt1, pt2, ehn, etn = outs

    e2r = _sc_edge_pass("gat", RH, NREL, NRELP, N, N, True, True, False, N)
    fh, sh = e2r(xrh, h_g, rel_s, h_g, t_g, ph1, ph2)
    ft, st = e2r(xrt, t_g, rel_s, h_g, t_g, pt1, pt2)
    x_r, r_proj = _tc_call(_xr_body, (
        jax.ShapeDtypeStruct((NREL, RH), f32),
        jax.ShapeDtypeStruct((NREL,), f32),
    ))(fh.reshape(NC, NRELP, RH), sh.reshape(NC, NRELP, L),
       ft.reshape(NC, NRELP, RH), st.reshape(NC, NRELP, L),
       r2e_ar.reshape(1, RH))

    fxh, sxh, fxt, sxt = _sc_r2e_dual()(
        x_r, rel_g, h_s, t_s, h_g, t_g, ehn, etn, r_proj)
    dcat = EH + 2 * RH
    xcat, gi, gj = pl.pallas_call(
        _cat_body, grid=(GN,),
        in_specs=[_rows(EH), _acc3(RH), _acc3(L), _acc3(RH), _acc3(L),
                  _full(1, dcat), _full(1, dcat)],
        out_specs=(_rows(dcat), _vec(), _vec()),
        out_shape=(
            jax.ShapeDtypeStruct((N, dcat), f32),
            jax.ShapeDtypeStruct((N,), f32),
            jax.ShapeDtypeStruct((N,), f32),
        ),
    )(x, fxh.reshape(NC, NP, RH), sxh.reshape(NC, NP, L),
      fxt.reshape(NC, NP, RH), sxt.reshape(NC, NP, L),
      gat_ai.reshape(1, dcat), gat_aj.reshape(1, dcat))

    fin_a = _sc_edge_pass("gat", 0, N, NP, N, N, False, True, True, 0)
    sg, w_all = fin_a(dst_a_s, dst_a_g, src_a_g, gi, gj)
    dh = dcat // 2
    fin_b = _sc_edge_pass("load", dh, N, NP, 0, 0, True, False, False, N)
    (fg0,) = fin_b(xcat[:, :dh], src_a_g, dst_a_s, w_all)
    (fg1,) = fin_b(xcat[:, dh:], src_a_g, dst_a_s, w_all)
    fg = jnp.concatenate([fg0.reshape(NC, NP, dh), fg1.reshape(NC, NP, dh)],
                         axis=2)

    return pl.pallas_call(
        _out_body, grid=(GN,),
        in_specs=[_rows(dcat), _acc3(dcat), _acc3(L)],
        out_specs=_rows(2 * dcat),
        out_shape=jax.ShapeDtypeStruct((N, 2 * dcat), f32))(
        xcat, fg, sg.reshape(NC, NP, L))


def kernel(x_e, edge_index, rel, edge_index_all, rel_all, hw1_W, hw1_b,
           hw2_W, hw2_b, e2r_ah1, e2r_ah2, e2r_at1, e2r_at2, e2r_wh,
           e2r_wt, r2e_ah, r2e_at, r2e_ar, gat_ai, gat_aj):
    return _run(x_e, edge_index, rel, edge_index_all,
                hw1_W, hw1_b, hw2_W, hw2_b,
                e2r_ah1, e2r_ah2, e2r_at1, e2r_at2, e2r_wh, e2r_wt,
                r2e_ah, r2e_at, r2e_ar, gat_ai, gat_aj)


# speedup vs baseline: 1.1951x; 1.0687x over previous
import functools

import jax
import jax.numpy as jnp
from jax import lax
from jax.experimental import pallas as pl
from jax.experimental.pallas import tpu as pltpu
from jax.experimental.pallas import tpu_sc as plsc

N = 10000
E = 640000
EH = 128
RH = 32
NREL = 1000

NC = 2
NS = 16
NW = NC * NS
L = 16

K = 128
NJ = 10
CHUNK = NJ * K
NCHUNK = 16
NBUF = 2
E_PER = NCHUNK * CHUNK
E_PAD = NW * E_PER

NP = 10240
NRELP = 1024

_MESH = dict(core_axis_name="c", subcore_axis_name="s", num_cores=NC,
             num_subcores=NS)


def _sc_edge_pass(mode, d, n_out, np_out, n_u, n_v, has_feat, has_sacc,
                  w_out, n_in, uv_same=False, i1_is_gidx=False):
    out_type = []
    if has_feat:
        out_type.append(jax.ShapeDtypeStruct((NC * np_out, d), jnp.float32))
    if has_sacc:
        out_type.append(jax.ShapeDtypeStruct((NC * np_out, L), jnp.float32))
    if w_out:
        out_type.append(jax.ShapeDtypeStruct((E_PAD,), jnp.float32))

    scratch = {}
    if has_feat:
        scratch["facc"] = pltpu.VMEM_SHARED((np_out, d), jnp.float32)
        scratch["gidx_v"] = pltpu.VMEM((CHUNK,), jnp.int32)
        scratch["rows_v"] = pltpu.VMEM((NBUF, K, d), jnp.float32)
    if has_sacc:
        scratch["sacc"] = pltpu.VMEM_SHARED((np_out, L), jnp.float32)
        scratch["brows_v"] = pltpu.VMEM((K, L), jnp.float32)
    scratch["sidx_v"] = pltpu.VMEM((NJ, K), jnp.int32)
    if mode in ("gat", "gcn"):
        if not i1_is_gidx:
            scratch["i1_v"] = pltpu.VMEM((CHUNK,), jnp.int32)
        scratch["i2_v"] = pltpu.VMEM((CHUNK,), jnp.int32)
        scratch["u_v"] = pltpu.VMEM((n_u,), jnp.float32)
        if not uv_same:
            scratch["v_v"] = pltpu.VMEM((n_v,), jnp.float32)
    if mode != "ones":
        scratch["w_v"] = pltpu.VMEM((CHUNK,), jnp.float32)
    scratch["sem"] = pltpu.SemaphoreType.DMA
    if has_feat:
        for _b in range(1, NBUF):
            scratch[f"sem{_b}"] = pltpu.SemaphoreType.DMA
        for _b in range(NBUF):
            scratch[f"ssem{_b}"] = pltpu.SemaphoreType.DMA
    snames = list(scratch.keys())

    def body(*refs):
        nin = 0
        args = {}
        if has_feat:
            args["x_hbm"] = refs[nin]; nin += 1
            args["gidx_hbm"] = refs[nin]; nin += 1
        args["sidx_hbm"] = refs[nin]; nin += 1
        if mode in ("gat", "gcn"):
            if not i1_is_gidx:
                args["i1_hbm"] = refs[nin]; nin += 1
            args["i2_hbm"] = refs[nin]; nin += 1
            args["u_hbm"] = refs[nin]; nin += 1
            if not uv_same:
                args["v_hbm"] = refs[nin]; nin += 1
        if mode == "load":
            args["wsrc_hbm"] = refs[nin]; nin += 1
        if has_feat:
            args["facc_out"] = refs[nin]; nin += 1
        if has_sacc:
            args["sacc_out"] = refs[nin]; nin += 1
        if w_out:
            args["w_hbm"] = refs[nin]; nin += 1
        for nm, r in zip(snames, refs[nin:]):
            args[nm] = r
        if mode in ("gat", "gcn"):
            if i1_is_gidx:
                args["i1_v"] = args["gidx_v"]
            if uv_same:
                args["v_v"] = args["u_v"]

        c = lax.axis_index("c")
        s = lax.axis_index("s")
        wid = c * NS + s

        if mode in ("gat", "gcn"):
            pltpu.sync_copy(args["u_hbm"], args["u_v"])
            if not uv_same:
                pltpu.sync_copy(args["v_hbm"], args["v_v"])

        zero16 = jnp.zeros((L,), jnp.float32)
        rows_per = np_out // NS
        base_rows = s * rows_per

        def zero_buf(zref, width):
            def zrow(i, _):
                for dd in range(width // L):
                    zref[i, pl.ds(dd * L, L)] = zero16
                return 0
            lax.fori_loop(0, K, zrow, 0)

        def zero_acc(zref, acc):
            zr = min(rows_per, K)
            def zstep(r, _):
                pltpu.sync_copy(zref.at[pl.ds(0, zr)],
                                acc.at[pl.ds(base_rows + r * zr, zr)])
                return 0
            lax.fori_loop(0, rows_per // zr, zstep, 0)

        if has_feat:
            zero_buf(args["rows_v"].at[0], d)
            zero_acc(args["rows_v"].at[0], args["facc"])
        if has_sacc:
            zero_buf(args["brows_v"], L)
            zero_acc(args["brows_v"], args["sacc"])
        plsc.subcore_barrier()

        if mode == "ones":
            one16 = jnp.full((L,), 1.0, jnp.float32)

            def orow(i, _):
                args["brows_v"][i, :] = one16
                return 0
            lax.fori_loop(0, K, orow, 0)

        sems = [args.get("sem")] + [args.get(f"sem{b}")
                                    for b in range(1, NBUF)]
        ssems = [args.get(f"ssem{b}") for b in range(NBUF)]

        def chunk_body(ci, _):
            ebase = (wid * NCHUNK + ci) * CHUNK
            rbase = (wid * NCHUNK + ci) * NJ
            if has_feat:
                pltpu.sync_copy(args["gidx_hbm"].at[pl.ds(ebase, CHUNK)],
                                args["gidx_v"])

            def gather(j, b):
                return pltpu.async_copy(
                    args["x_hbm"].at[args["gidx_v"].at[pl.ds(j * K, K)]],
                    args["rows_v"].at[b], sems[b])

            cps = {}
            if has_feat:
                cps[0] = gather(0, 0)
            pltpu.sync_copy(args["sidx_hbm"].at[pl.ds(rbase, NJ)],
                            args["sidx_v"])
            if mode in ("gat", "gcn"):
                if not i1_is_gidx:
                    pltpu.sync_copy(args["i1_hbm"].at[pl.ds(ebase, CHUNK)],
                                    args["i1_v"])
                pltpu.sync_copy(args["i2_hbm"].at[pl.ds(ebase, CHUNK)],
                                args["i2_v"])

                def wstep(i, _):
                    off = i * L
                    idx1 = args["i1_v"][pl.ds(off, L)]
                    idx2 = args["i2_v"][pl.ds(off, L)]
                    a = plsc.load_gather(args["u_v"], [idx1])
                    b = plsc.load_gather(args["v_v"], [idx2])
                    if mode == "gat":
                        z = a + b
                        w16 = jnp.exp(jnp.maximum(z, 0.01 * z))
                    else:
                        w16 = a * b
                    args["w_v"][pl.ds(off, L)] = w16
                    return 0
                lax.fori_loop(0, CHUNK // L, wstep, 0)
            if mode == "load":
                pltpu.sync_copy(args["wsrc_hbm"].at[pl.ds(ebase, CHUNK)],
                                args["w_v"])
            if w_out:
                pltpu.sync_copy(args["w_v"],
                                args["w_hbm"].at[pl.ds(ebase, CHUNK)])

            scat = {}
            for j in range(NJ):
                b = j % NBUF
                if has_feat:
                    cps[b].wait()
                    if j + 1 < NJ:
                        b2 = (j + 1) % NBUF
                        if b2 in scat:
                            scat[b2].wait()
                        cps[b2] = gather(j + 1, b2)
                    rbuf = args["rows_v"].at[b]
                if mode != "ones":
                    def scale_grp(ii, _):
                        gbase = ii * L
                        w16 = args["w_v"][pl.ds(j * K + gbase, L)]
                        for l in range(L):
                            wb = lax.broadcast(w16[l], (L,))
                            row = gbase + l
                            if has_feat:
                                vals = [rbuf[row, pl.ds(dd * L, L)]
                                        for dd in range(d // L)]
                                for dd in range(d // L):
                                    rbuf[row, pl.ds(dd * L, L)] = (
                                        vals[dd] * wb)
                            if has_sacc:
                                args["brows_v"][row, :] = wb
                        return 0
                    lax.fori_loop(0, K // L, scale_grp, 0)
                row_idx = args["sidx_v"].at[j]
                if has_feat:
                    scat[b] = pltpu.async_copy(
                        rbuf, args["facc"].at[row_idx], ssems[b], add=True)
                if has_sacc:
                    pltpu.sync_copy(args["brows_v"],
                                    args["sacc"].at[row_idx], add=True)
            for b in list(scat):
                scat[b].wait()
            return 0
        lax.fori_loop(0, NCHUNK, chunk_body, 0)

        plsc.subcore_barrier()
        obase = c * np_out + base_rows
        if has_feat:
            pltpu.sync_copy(args["facc"].at[pl.ds(base_rows, rows_per)],
                            args["facc_out"].at[pl.ds(obase, rows_per)])
        if has_sacc:
            pltpu.sync_copy(args["sacc"].at[pl.ds(base_rows, rows_per)],
                            args["sacc_out"].at[pl.ds(obase, rows_per)])

    mesh = plsc.VectorSubcoreMesh(**_MESH)
    return pl.kernel(body, out_type=tuple(out_type), mesh=mesh,
                     scratch_types=list(scratch.values()),
                     compiler_params=pltpu.CompilerParams(
                         needs_layout_passes=False,
                         use_tc_tiling_on_sc=False))


def _sc_r2e_dual():
    d = RH
    out_type = (
        jax.ShapeDtypeStruct((NC * NP, d), jnp.float32),
        jax.ShapeDtypeStruct((NC * NP, L), jnp.float32),
        jax.ShapeDtypeStruct((NC * NP, d), jnp.float32),
        jax.ShapeDtypeStruct((NC * NP, L), jnp.float32),
    )
    scratch = {
        "facc1": pltpu.VMEM_SHARED((NP, d), jnp.float32),
        "sacc1": pltpu.VMEM_SHARED((NP, L), jnp.float32),
        "facc2": pltpu.VMEM_SHARED((NP, d), jnp.float32),
        "sacc2": pltpu.VMEM_SHARED((NP, L), jnp.float32),
        "gidx_v": pltpu.VMEM((CHUNK,), jnp.int32),
        "h_v": pltpu.VMEM((CHUNK,), jnp.int32),
        "t_v": pltpu.VMEM((CHUNK,), jnp.int32),
        "sidx1_v": pltpu.VMEM((NJ, K), jnp.int32),
        "sidx2_v": pltpu.VMEM((NJ, K), jnp.int32),
        "u1_v": pltpu.VMEM((N,), jnp.float32),
        "u2_v": pltpu.VMEM((N,), jnp.float32),
        "v_v": pltpu.VMEM((NREL,), jnp.float32),
        "w1_v": pltpu.VMEM((CHUNK,), jnp.float32),
        "w2_v": pltpu.VMEM((CHUNK,), jnp.float32),
        "raw_v": pltpu.VMEM((NBUF, K, d), jnp.float32),
        "s1_v": pltpu.VMEM((NBUF, K, d), jnp.float32),
        "s2_v": pltpu.VMEM((NBUF, K, d), jnp.float32),
        "brows1_v": pltpu.VMEM((K, L), jnp.float32),
        "brows2_v": pltpu.VMEM((K, L), jnp.float32),
    }
    for _b in range(NBUF):
        scratch[f"gsem{_b}"] = pltpu.SemaphoreType.DMA
        scratch[f"s1sem{_b}"] = pltpu.SemaphoreType.DMA
        scratch[f"s2sem{_b}"] = pltpu.SemaphoreType.DMA
    snames = list(scratch.keys())

    def body(xr_hbm, rel_hbm, h2d_hbm, t2d_hbm, h_hbm, t_hbm,
             u1_hbm, u2_hbm, v_hbm, facc1_out, sacc1_out, facc2_out,
             sacc2_out, *srefs):
        a = dict(zip(snames, srefs))
        c = lax.axis_index("c")
        s = lax.axis_index("s")
        wid = c * NS + s
        pltpu.sync_copy(u1_hbm, a["u1_v"])
        pltpu.sync_copy(u2_hbm, a["u2_v"])
        pltpu.sync_copy(v_hbm, a["v_v"])

        zero16 = jnp.zeros((L,), jnp.float32)
        rows_per = NP // NS
        base_rows = s * rows_per

        def zero_buf(zref, width):
            def zrow(i, _):
                for dd in range(width // L):
                    zref[i, pl.ds(dd * L, L)] = zero16
                return 0
            lax.fori_loop(0, K, zrow, 0)

        def zero_acc(zref, acc):
            def zstep(r, _):
                pltpu.sync_copy(zref.at[pl.ds(0, K)],
                                acc.at[pl.ds(base_rows + r * K, K)])
                return 0
            lax.fori_loop(0, rows_per // K, zstep, 0)

        zero_buf(a["s1_v"].at[0], d)
        zero_buf(a["brows1_v"], L)
        zero_acc(a["s1_v"].at[0], a["facc1"])
        zero_acc(a["s1_v"].at[0], a["facc2"])
        zero_acc(a["brows1_v"], a["sacc1"])
        zero_acc(a["brows1_v"], a["sacc2"])
        plsc.subcore_barrier()

        gsems = [a[f"gsem{b}"] for b in range(NBUF)]
        s1sems = [a[f"s1sem{b}"] for b in range(NBUF)]
        s2sems = [a[f"s2sem{b}"] for b in range(NBUF)]

        def chunk_body(ci, _):
            ebase = (wid * NCHUNK + ci) * CHUNK
            rbase = (wid * NCHUNK + ci) * NJ
            pltpu.sync_copy(rel_hbm.at[pl.ds(ebase, CHUNK)], a["gidx_v"])

            def gather(j, b):
                return pltpu.async_copy(
                    xr_hbm.at[a["gidx_v"].at[pl.ds(j * K, K)]],
                    a["raw_v"].at[b], gsems[b])

            cps = {0: gather(0, 0)}
            pltpu.sync_copy(h2d_hbm.at[pl.ds(rbase, NJ)], a["sidx1_v"])
            pltpu.sync_copy(t2d_hbm.at[pl.ds(rbase, NJ)], a["sidx2_v"])
            pltpu.sync_copy(h_hbm.at[pl.ds(ebase, CHUNK)], a["h_v"])
            pltpu.sync_copy(t_hbm.at[pl.ds(ebase, CHUNK)], a["t_v"])

            def wstep(i, _):
                off = i * L
                ih = a["h_v"][pl.ds(off, L)]
                it = a["t_v"][pl.ds(off, L)]
                ir = a["gidx_v"][pl.ds(off, L)]
                a1 = plsc.load_gather(a["u1_v"], [ih])
                a2 = plsc.load_gather(a["u2_v"], [it])
                br = plsc.load_gather(a["v_v"], [ir])
                z1 = a1 + br
                z2 = a2 + br
                a["w1_v"][pl.ds(off, L)] = jnp.exp(
                    jnp.maximum(z1, 0.01 * z1))
                a["w2_v"][pl.ds(off, L)] = jnp.exp(
                    jnp.maximum(z2, 0.01 * z2))
                return 0
            lax.fori_loop(0, CHUNK // L, wstep, 0)

            sc1, sc2 = {}, {}
            for j in range(NJ):
                b = j % NBUF
                cps[b].wait()
                if j + 1 < NJ:
                    b2 = (j + 1) % NBUF
                    cps[b2] = gather(j + 1, b2)
                if b in sc1:
                    sc1[b].wait()
                    sc2[b].wait()
                rraw = a["raw_v"].at[b]
                rs1 = a["s1_v"].at[b]
                rs2 = a["s2_v"].at[b]

                def scale_grp(ii, _):
                    gbase = ii * L
                    w116 = a["w1_v"][pl.ds(j * K + gbase, L)]
                    w216 = a["w2_v"][pl.ds(j * K + gbase, L)]
                    for l in range(L):
                        wb1 = lax.broadcast(w116[l], (L,))
                        wb2 = lax.broadcast(w216[l], (L,))
                        row = gbase + l
                        vals = [rraw[row, pl.ds(dd * L, L)]
                                for dd in range(d // L)]
                        for dd in range(d // L):
                            rs1[row, pl.ds(dd * L, L)] = vals[dd] * wb1
                        for dd in range(d // L):
                            rs2[row, pl.ds(dd * L, L)] = vals[dd] * wb2
                        a["brows1_v"][row, :] = wb1
                        a["brows2_v"][row, :] = wb2
                    return 0
                lax.fori_loop(0, K // L, scale_grp, 0)

                i1 = a["sidx1_v"].at[j]
                i2 = a["sidx2_v"].at[j]
                sc1[b] = pltpu.async_copy(rs1, a["facc1"].at[i1],
                                          s1sems[b], add=True)
                sc2[b] = pltpu.async_copy(rs2, a["facc2"].at[i2],
                                          s2sems[b], add=True)
                pltpu.sync_copy(a["brows1_v"], a["sacc1"].at[i1], add=True)
                pltpu.sync_copy(a["brows2_v"], a["sacc2"].at[i2], add=True)
            for b in list(sc1):
                sc1[b].wait()
                sc2[b].wait()
            return 0
        lax.fori_loop(0, NCHUNK, chunk_body, 0)

        plsc.subcore_barrier()
        obase = c * NP + base_rows
        sl_s = pl.ds(base_rows, rows_per)
        sl_o = pl.ds(obase, rows_per)
        pltpu.sync_copy(a["facc1"].at[sl_s], facc1_out.at[sl_o])
        pltpu.sync_copy(a["sacc1"].at[sl_s], sacc1_out.at[sl_o])
        pltpu.sync_copy(a["facc2"].at[sl_s], facc2_out.at[sl_o])
        pltpu.sync_copy(a["sacc2"].at[sl_s], sacc2_out.at[sl_o])

    mesh = plsc.VectorSubcoreMesh(**_MESH)
    return pl.kernel(body, out_type=out_type, mesh=mesh,
                     scratch_types=list(scratch.values()),
                     compiler_params=pltpu.CompilerParams(
                         needs_layout_passes=False,
                         use_tc_tiling_on_sc=False))


def _sc_e2r_dual():
    d = RH
    out_type = (
        jax.ShapeDtypeStruct((NC * NRELP, d), jnp.float32),
        jax.ShapeDtypeStruct((NC * NRELP, L), jnp.float32),
        jax.ShapeDtypeStruct((NC * NRELP, d), jnp.float32),
        jax.ShapeDtypeStruct((NC * NRELP, L), jnp.float32),
    )
    scratch = {
        "facc1": pltpu.VMEM_SHARED((NRELP, d), jnp.float32),
        "sacc1": pltpu.VMEM_SHARED((NRELP, L), jnp.float32),
        "facc2": pltpu.VMEM_SHARED((NRELP, d), jnp.float32),
        "sacc2": pltpu.VMEM_SHARED((NRELP, L), jnp.float32),
        "h_v": pltpu.VMEM((CHUNK,), jnp.int32),
        "t_v": pltpu.VMEM((CHUNK,), jnp.int32),
        "sidx_v": pltpu.VMEM((NJ, K), jnp.int32),
        "u1_v": pltpu.VMEM((N,), jnp.float32),
        "u2_v": pltpu.VMEM((N,), jnp.float32),
        "u3_v": pltpu.VMEM((N,), jnp.float32),
        "u4_v": pltpu.VMEM((N,), jnp.float32),
        "w1_v": pltpu.VMEM((CHUNK,), jnp.float32),
        "w2_v": pltpu.VMEM((CHUNK,), jnp.float32),
        "r1_v": pltpu.VMEM((NBUF, K, d), jnp.float32),
        "r2_v": pltpu.VMEM((NBUF, K, d), jnp.float32),
        "brows1_v": pltpu.VMEM((K, L), jnp.float32),
        "brows2_v": pltpu.VMEM((K, L), jnp.float32),
    }
    for _b in range(NBUF):
        scratch[f"g1sem{_b}"] = pltpu.SemaphoreType.DMA
        scratch[f"g2sem{_b}"] = pltpu.SemaphoreType.DMA
        scratch[f"s1sem{_b}"] = pltpu.SemaphoreType.DMA
        scratch[f"s2sem{_b}"] = pltpu.SemaphoreType.DMA
    snames = list(scratch.keys())

    def body(xrh_hbm, xrt_hbm, h_hbm, t_hbm, rel2d_hbm,
             u1_hbm, u2_hbm, u3_hbm, u4_hbm,
             facc1_out, sacc1_out, facc2_out, sacc2_out, *srefs):
        a = dict(zip(snames, srefs))
        c = lax.axis_index("c")
        s = lax.axis_index("s")
        wid = c * NS + s
        pltpu.sync_copy(u1_hbm, a["u1_v"])
        pltpu.sync_copy(u2_hbm, a["u2_v"])
        pltpu.sync_copy(u3_hbm, a["u3_v"])
        pltpu.sync_copy(u4_hbm, a["u4_v"])

        zero16 = jnp.zeros((L,), jnp.float32)
        rows_per = NRELP // NS
        base_rows = s * rows_per

        def zero_buf(zref, width, nrow):
            def zrow(i, _):
                for dd in range(width // L):
                    zref[i, pl.ds(dd * L, L)] = zero16
                return 0
            lax.fori_loop(0, nrow, zrow, 0)

        def zero_acc(zref, acc):
            zr = min(rows_per, K)
            def zstep(r, _):
                pltpu.sync_copy(zref.at[pl.ds(0, zr)],
                                acc.at[pl.ds(base_rows + r * zr, zr)])
                return 0
            lax.fori_loop(0, rows_per // zr, zstep, 0)

        zero_buf(a["r1_v"].at[0], d, K)
        zero_buf(a["brows1_v"], L, K)
        zero_acc(a["r1_v"].at[0], a["facc1"])
        zero_acc(a["r1_v"].at[0], a["facc2"])
        zero_acc(a["brows1_v"], a["sacc1"])
        zero_acc(a["brows1_v"], a["sacc2"])
        plsc.subcore_barrier()

        g1sems = [a[f"g1sem{b}"] for b in range(NBUF)]
        g2sems = [a[f"g2sem{b}"] for b in range(NBUF)]
        s1sems = [a[f"s1sem{b}"] for b in range(NBUF)]
        s2sems = [a[f"s2sem{b}"] for b in range(NBUF)]

        def chunk_body(ci, _):
            ebase = (wid * NCHUNK + ci) * CHUNK
            rbase = (wid * NCHUNK + ci) * NJ
            pltpu.sync_copy(h_hbm.at[pl.ds(ebase, CHUNK)], a["h_v"])
            pltpu.sync_copy(t_hbm.at[pl.ds(ebase, CHUNK)], a["t_v"])

            def gather1(j, b):
                return pltpu.async_copy(
                    xrh_hbm.at[a["h_v"].at[pl.ds(j * K, K)]],
                    a["r1_v"].at[b], g1sems[b])

            def gather2(j, b):
                return pltpu.async_copy(
                    xrt_hbm.at[a["t_v"].at[pl.ds(j * K, K)]],
                    a["r2_v"].at[b], g2sems[b])

            cp1 = {0: gather1(0, 0)}
            cp2 = {0: gather2(0, 0)}
            pltpu.sync_copy(rel2d_hbm.at[pl.ds(rbase, NJ)], a["sidx_v"])

            def wstep(i, _):
                off = i * L
                ih = a["h_v"][pl.ds(off, L)]
                it = a["t_v"][pl.ds(off, L)]
                a1 = plsc.load_gather(a["u1_v"], [ih])
                b1 = plsc.load_gather(a["u2_v"], [it])
                a2 = plsc.load_gather(a["u3_v"], [ih])
                b2 = plsc.load_gather(a["u4_v"], [it])
                z1 = a1 + b1
                z2 = a2 + b2
                a["w1_v"][pl.ds(off, L)] = jnp.exp(
                    jnp.maximum(z1, 0.01 * z1))
                a["w2_v"][pl.ds(off, L)] = jnp.exp(
                    jnp.maximum(z2, 0.01 * z2))
                return 0
            lax.fori_loop(0, CHUNK // L, wstep, 0)

            sc1, sc2 = {}, {}
            for j in range(NJ):
                b = j % NBUF
                cp1[b].wait()
                cp2[b].wait()
                if b in sc1:
                    sc1[b].wait()
                    sc2[b].wait()
                if j + 1 < NJ:
                    b2 = (j + 1) % NBUF
                    if b2 in sc1:
                        sc1[b2].wait()
                        del sc1[b2]
                        sc2[b2].wait()
                        del sc2[b2]
                    cp1[b2] = gather1(j + 1, b2)
                    cp2[b2] = gather2(j + 1, b2)
                rs1 = a["r1_v"].at[b]
                rs2 = a["r2_v"].at[b]

                def scale_grp(ii, _):
                    gbase = ii * L
                    w116 = a["w1_v"][pl.ds(j * K + gbase, L)]
                    w216 = a["w2_v"][pl.ds(j * K + gbase, L)]
                    for l in range(L):
                        wb1 = lax.broadcast(w116[l], (L,))
                        wb2 = lax.broadcast(w216[l], (L,))
                        row = gbase + l
                        v1 = [rs1[row, pl.ds(dd * L, L)]
                              for dd in range(d // L)]
                        v2 = [rs2[row, pl.ds(dd * L, L)]
                              for dd in range(d // L)]
                        for dd in range(d // L):
                            rs1[row, pl.ds(dd * L, L)] = v1[dd] * wb1
                        for dd in range(d // L):
                            rs2[row, pl.ds(dd * L, L)] = v2[dd] * wb2
                        a["brows1_v"][row, :] = wb1
                        a["brows2_v"][row, :] = wb2
                    return 0
                lax.fori_loop(0, K // L, scale_grp, 0)

                ri = a["sidx_v"].at[j]
                sc1[b] = pltpu.async_copy(rs1, a["facc1"].at[ri],
                                          s1sems[b], add=True)
                sc2[b] = pltpu.async_copy(rs2, a["facc2"].at[ri],
                                          s2sems[b], add=True)
                pltpu.sync_copy(a["brows1_v"], a["sacc1"].at[ri], add=True)
                pltpu.sync_copy(a["brows2_v"], a["sacc2"].at[ri], add=True)
            for b in list(sc1):
                sc1[b].wait()
                sc2[b].wait()
            return 0
        lax.fori_loop(0, NCHUNK, chunk_body, 0)

        plsc.subcore_barrier()
        obase = c * NRELP + base_rows
        sl_s = pl.ds(base_rows, rows_per)
        sl_o = pl.ds(obase, rows_per)
        pltpu.sync_copy(a["facc1"].at[sl_s], facc1_out.at[sl_o])
        pltpu.sync_copy(a["sacc1"].at[sl_s], sacc1_out.at[sl_o])
        pltpu.sync_copy(a["facc2"].at[sl_s], facc2_out.at[sl_o])
        pltpu.sync_copy(a["sacc2"].at[sl_s], sacc2_out.at[sl_o])

    mesh = plsc.VectorSubcoreMesh(**_MESH)
    return pl.kernel(body, out_type=out_type, mesh=mesh,
                     scratch_types=list(scratch.values()),
                     compiler_params=pltpu.CompilerParams(
                         needs_layout_passes=False,
                         use_tc_tiling_on_sc=False))


R = 2048
GN = 5


def _tc_call(body, out_type):
    return pl.pallas_call(body, out_shape=out_type)


def _rows(w):
    return pl.BlockSpec((R, w), lambda i: (i, 0))


def _acc3(w):
    return pl.BlockSpec((NC, R, w), lambda i: (0, i, 0))


def _full(*shape):
    nd = len(shape)
    return pl.BlockSpec(shape, lambda i: (0,) * nd)


def _vec():
    return pl.BlockSpec((R,), lambda i: (i,))


def _inv0(s3):
    s0 = s3[0, :, 0] + s3[1, :, 0]
    return jnp.where(s0 > 0, 1.0 / s0, 0.0)[:, None]


def _dis_body(sacc_ref, out_ref):
    a = sacc_ref[...]
    deg = a[0, :, 0] + a[1, :, 0]
    out_ref[...] = jnp.where(deg > 0, lax.rsqrt(jnp.maximum(deg, 1e-30)), 0.0)


def _hw_body(xin_ref, gp_ref, w_ref, b_ref, out_ref):
    gp = gp_ref[...]
    g = jax.nn.relu(gp[0] + gp[1])
    xin = xin_ref[...]
    gate = jax.nn.sigmoid(
        jnp.dot(xin, w_ref[...], preferred_element_type=jnp.float32)
        + b_ref[...])
    out_ref[...] = gate * g + (1.0 - gate) * xin


def _proj_body(x_ref, wh_ref, wt_ref, ah1_ref, ah2_ref, at1_ref, at2_ref,
               rah_ref, rat_ref,
               xrh_ref, xrt_ref, ph1_ref, ph2_ref, pt1_ref, pt2_ref,
               ehn_ref, etn_ref):
    x = x_ref[...]
    xrh = jnp.dot(x, wh_ref[...], preferred_element_type=jnp.float32)
    xrt = jnp.dot(x, wt_ref[...], preferred_element_type=jnp.float32)
    xrh_ref[...] = xrh
    xrt_ref[...] = xrt
    ph1_ref[...] = jnp.sum(xrh * ah1_ref[...], axis=1)
    ph2_ref[...] = jnp.sum(xrt * ah2_ref[...], axis=1)
    pt1_ref[...] = jnp.sum(xrh * at1_ref[...], axis=1)
    pt2_ref[...] = jnp.sum(xrt * at2_ref[...], axis=1)
    ehn_ref[...] = jnp.sum(x * rah_ref[...], axis=1)
    etn_ref[...] = jnp.sum(x * rat_ref[...], axis=1)


def _xr_body(fh_ref, sh_ref, ft_ref, st_ref, ar_ref, xr_ref, rp_ref):
    fh = fh_ref[...]
    ft = ft_ref[...]
    sh = sh_ref[...]
    st = st_ref[...]
    sh0 = sh[0, :NREL, 0] + sh[1, :NREL, 0]
    st0 = st[0, :NREL, 0] + st[1, :NREL, 0]
    inv_h = jnp.where(sh0 > 0, 1.0 / sh0, 0.0)[:, None]
    inv_t = jnp.where(st0 > 0, 1.0 / st0, 0.0)[:, None]
    xr = (fh[0, :NREL, :] + fh[1, :NREL, :]) * inv_h \
        + (ft[0, :NREL, :] + ft[1, :NREL, :]) * inv_t
    xr_ref[...] = xr
    rp_ref[...] = jnp.sum(xr * ar_ref[...], axis=1)


def _cat_body(x_ref, fh_ref, sh_ref, ft_ref, st_ref, ai_ref, aj_ref,
              xcat_ref, gi_ref, gj_ref):
    fh = fh_ref[...]
    ft = ft_ref[...]
    xeh = (fh[0] + fh[1]) * _inv0(sh_ref[...])
    xet = (ft[0] + ft[1]) * _inv0(st_ref[...])
    xcat = jnp.concatenate([x_ref[...], xeh, xet], axis=1)
    xcat_ref[...] = xcat
    gi_ref[...] = jnp.sum(xcat * ai_ref[...], axis=1)
    gj_ref[...] = jnp.sum(xcat * aj_ref[...], axis=1)


def _out_body(xcat_ref, fg_ref, sg_ref, out_ref):
    fg = fg_ref[...]
    xg = jax.nn.relu((fg[0] + fg[1]) * _inv0(sg_ref[...]))
    out_ref[...] = jnp.concatenate([xcat_ref[...], xg], axis=1)


def _padi(a, fill):
    return jnp.concatenate(
        [a, jnp.full((E_PAD - E,), fill, a.dtype)])


@jax.jit
def _run(x_e, edge_index, rel, edge_index_all,
         hw1_W, hw1_b, hw2_W, hw2_b,
         e2r_ah1, e2r_ah2, e2r_at1, e2r_at2, e2r_wh, e2r_wt,
         r2e_ah, r2e_at, r2e_ar, gat_ai, gat_aj):
    f32 = jnp.float32
    src_a = edge_index_all[0]
    dst_a = edge_index_all[1]
    h = edge_index[0]
    t = edge_index[1]

    src_a_g = _padi(src_a, 0)
    dst_a_g = _padi(dst_a, 0)
    dst_a_s = _padi(dst_a, N).reshape(E_PAD // K, K)
    h_g = _padi(h, 0)
    t_g = _padi(t, 0)
    rel_g = _padi(rel, 0)
    h_s = _padi(h, N).reshape(E_PAD // K, K)
    t_s = _padi(t, N).reshape(E_PAD // K, K)
    rel_s = _padi(rel, NREL).reshape(E_PAD // K, K)

    deg_pass = _sc_edge_pass("ones", 0, N, NP, 0, 0, False, True, False, 0)
    (sacc_deg,) = deg_pass(dst_a_s)
    dis = pl.pallas_call(
        _dis_body, grid=(GN,), in_specs=[_acc3(L)], out_specs=_vec(),
        out_shape=jax.ShapeDtypeStruct((N,), f32))(
        sacc_deg.reshape(NC, NP, L))

    gcn = _sc_edge_pass("gcn", EH, N, NP, N, N, True, False, False, N,
                        uv_same=True, i1_is_gidx=True)
    (g1,) = gcn(x_e, src_a_g, dst_a_s, dst_a_g, dis)
    hw_call = pl.pallas_call(
        _hw_body, grid=(GN,),
        in_specs=[_rows(EH), _acc3(EH), _full(EH, EH), _full(1, EH)],
        out_specs=_rows(EH),
        out_shape=jax.ShapeDtypeStruct((N, EH), f32))
    x1 = hw_call(x_e, g1.reshape(NC, NP, EH), hw1_W, hw1_b.reshape(1, EH))

    (g2,) = gcn(x1, src_a_g, dst_a_s, dst_a_g, dis)
    x = hw_call(x1, g2.reshape(NC, NP, EH), hw2_W, hw2_b.reshape(1, EH))

    outs = pl.pallas_call(
        _proj_body, grid=(GN,),
        in_specs=[_rows(EH), _full(EH, RH), _full(EH, RH)]
        + [_full(1, RH)] * 4 + [_full(1, EH)] * 2,
        out_specs=(_rows(RH), _rows(RH)) + (_vec(),) * 6,
        out_shape=(
            jax.ShapeDtypeStruct((N, RH), f32),
            jax.ShapeDtypeStruct((N, RH), f32),
        ) + (jax.ShapeDtypeStruct((N,), f32),) * 6,
    )(x, e2r_wh, e2r_wt,
      e2r_ah1.reshape(1, RH), e2r_ah2.reshape(1, RH),
      e2r_at1.reshape(1, RH), e2r_at2.reshape(1, RH),
      r2e_ah.reshape(1, EH), r2e_at.reshape(1, EH))
    xrh, xrt, ph1, ph2, pt1, pt2, ehn, etn = outs

    fh, sh, ft, st = _sc_e2r_dual()(
        xrh, xrt, h_g, t_g, rel_s, ph1, ph2, pt1, pt2)
    x_r, r_proj = _tc_call(_xr_body, (
        jax.ShapeDtypeStruct((NREL, RH), f32),
        jax.ShapeDtypeStruct((NREL,), f32),
    ))(fh.reshape(NC, NRELP, RH), sh.reshape(NC, NRELP, L),
       ft.reshape(NC, NRELP, RH), st.reshape(NC, NRELP, L),
       r2e_ar.reshape(1, RH))

    fxh, sxh, fxt, sxt = _sc_r2e_dual()(
        x_r, rel_g, h_s, t_s, h_g, t_g, ehn, etn, r_proj)
    dcat = EH + 2 * RH
    xcat, gi, gj = pl.pallas_call(
        _cat_body, grid=(GN,),
        in_specs=[_rows(EH), _acc3(RH), _acc3(L), _acc3(RH), _acc3(L),
                  _full(1, dcat), _full(1, dcat)],
        out_specs=(_rows(dcat), _vec(), _vec()),
        out_shape=(
            jax.ShapeDtypeStruct((N, dcat), f32),
            jax.ShapeDtypeStruct((N,), f32),
            jax.ShapeDtypeStruct((N,), f32),
        ),
    )(x, fxh.reshape(NC, NP, RH), sxh.reshape(NC, NP, L),
      fxt.reshape(NC, NP, RH), sxt.reshape(NC, NP, L),
      gat_ai.reshape(1, dcat), gat_aj.reshape(1, dcat))

    fin_a = _sc_edge_pass("gat", 0, N, NP, N, N, False, True, True, 0)
    sg, w_all = fin_a(dst_a_s, dst_a_g, src_a_g, gi, gj)
    dh = dcat // 2
    fin_b = _sc_edge_pass("load", dh, N, NP, 0, 0, True, False, False, N)
    (fg0,) = fin_b(xcat[:, :dh], src_a_g, dst_a_s, w_all)
    (fg1,) = fin_b(xcat[:, dh:], src_a_g, dst_a_s, w_all)
    fg = jnp.concatenate([fg0.reshape(NC, NP, dh), fg1.reshape(NC, NP, dh)],
                         axis=2)

    return pl.pallas_call(
        _out_body, grid=(GN,),
        in_specs=[_rows(dcat), _acc3(dcat), _acc3(L)],
        out_specs=_rows(2 * dcat),
        out_shape=jax.ShapeDtypeStruct((N, 2 * dcat), f32))(
        xcat, fg, sg.reshape(NC, NP, L))


def kernel(x_e, edge_index, rel, edge_index_all, rel_all, hw1_W, hw1_b,
           hw2_W, hw2_b, e2r_ah1, e2r_ah2, e2r_at1, e2r_at2, e2r_wh,
           e2r_wt, r2e_ah, r2e_at, r2e_ar, gat_ai, gat_aj):
    return _run(x_e, edge_index, rel, edge_index_all,
                hw1_W, hw1_b, hw2_W, hw2_b,
                e2r_ah1, e2r_ah2, e2r_at1, e2r_at2, e2r_wh, e2r_wt,
                r2e_ah, r2e_at, r2e_ar, gat_ai, gat_aj)
